# baseline (speedup 1.0000x reference)
import functools

import jax
import jax.numpy as jnp
from jax import lax
from jax.experimental import pallas as pl
from jax.experimental.pallas import tpu as pltpu
from jax.experimental.pallas import tpu_sc as plsc

N = 100000
C = 64
K = 27
BLK = 1024
NB = 98
NP = NB * BLK
GCH = 128
NCH = NP // GCH
ECH = 2000
KS0 = 14


def _sc_invert_gather(feats_pad, src, dst):
  mesh = plsc.VectorSubcoreMesh(core_axis_name="c", subcore_axis_name="s")

  @functools.partial(
      pl.kernel,
      out_type=(jax.ShapeDtypeStruct((NB, K, BLK, C), jnp.float32),
                jax.ShapeDtypeStruct((K * NP,), jnp.int32)),
      mesh=mesh,
      compiler_params=pltpu.CompilerParams(
          needs_layout_passes=False, use_tc_tiling_on_sc=False),
      scratch_types=[
          pltpu.VMEM((NP,), jnp.int32),
          pltpu.VMEM((ECH,), jnp.int32),
          pltpu.VMEM((ECH,), jnp.int32),
          pltpu.VMEM((GCH,), jnp.int32),
          pltpu.VMEM((GCH, C), jnp.float32),
          pltpu.SemaphoreType.DMA,
      ],
  )
  def sc_kernel(feats_hbm, src_hbm, dst_hbm, g_hbm, nbr_hbm,
                nbr_v, dbuf, sbuf, idx_v, rows_v, sem):
    cid = lax.axis_index("c")
    sid = lax.axis_index("s")
    kbase = cid * KS0
    nk = KS0 - cid

    @pl.when(sid < nk)
    def _build():
      k = kbase + sid

      @pl.loop(0, NP // 16)
      def _init(i):
        nbr_v[pl.ds(i * 16, 16)] = jnp.full((16,), N, jnp.int32)

      @pl.loop(0, N // ECH)
      def _chunk(j):
        e0 = pl.multiple_of(k * N + j * ECH, 8)
        pltpu.sync_copy(dst_hbm.at[pl.ds(e0, ECH)], dbuf)
        pltpu.sync_copy(src_hbm.at[pl.ds(e0, ECH)], sbuf)

        @pl.loop(0, ECH // 16)
        def _scatter(i):
          dv = dbuf[pl.ds(i * 16, 16)]
          sv = sbuf[pl.ds(i * 16, 16)]
          plsc.store_scatter(nbr_v, [dv], sv)

      nbr_v[pl.ds(N, 16)] = jnp.full((16,), N, jnp.int32)
      pltpu.sync_copy(nbr_v, nbr_hbm.at[pl.ds(pl.multiple_of(k * NP, 8), NP)])

    plsc.subcore_barrier()

    for k_i in range(KS0):
      @pl.when(k_i < nk)
      def _gather(k_i=k_i):
        k = kbase + k_i

        @pl.loop(0, NCH // 16)
        def _chunk(j):
          c = j * 16 + sid
          r0 = c * GCH
          b = r0 // BLK
          ri = pl.multiple_of(r0 - b * BLK, 8)
          sp0 = pl.multiple_of(k * NP + r0, 8)
          pltpu.sync_copy(nbr_hbm.at[pl.ds(sp0, GCH)], idx_v)
          pltpu.async_copy(feats_hbm.at[idx_v], rows_v, sem).wait()
          pltpu.sync_copy(rows_v, g_hbm.at[b, k, pl.ds(ri, GCH)])

  return sc_kernel(feats_pad, src, dst)[0]


def _tc_gemm(g, wflat):
  def body(g_ref, w_ref, o_ref, x_ref):
    for k in range(K):
      x_ref[:, k * C:(k + 1) * C] = g_ref[0, k, :, :]
    o_ref[...] = jnp.dot(x_ref[...], w_ref[...],
                         preferred_element_type=jnp.float32)

  return pl.pallas_call(
      body,
      grid=(NB,),
      in_specs=[
          pl.BlockSpec((1, K, BLK, C), lambda b: (b, 0, 0, 0)),
          pl.BlockSpec((K * C, C), lambda b: (0, 0)),
      ],
      out_specs=pl.BlockSpec((BLK, C), lambda b: (b, 0)),
      out_shape=jax.ShapeDtypeStruct((NP, C), jnp.float32),
      scratch_shapes=[pltpu.VMEM((BLK, K * C), jnp.float32)],
      compiler_params=pltpu.CompilerParams(
          dimension_semantics=("parallel",)),
  )(g, wflat)


def _tc_stats(out_pre):
  def body(o_ref, st_ref):
    x = o_ref[...]
    s = jnp.sum(x, axis=0, keepdims=True)
    q = jnp.sum(x * x, axis=0, keepdims=True)
    z = jnp.zeros((1, C), jnp.float32)
    tile = jnp.concatenate(
        [jnp.concatenate([s, z], axis=1),
         jnp.concatenate([q, z], axis=1),
         jnp.zeros((6, 128), jnp.float32)], axis=0)

    @pl.when(pl.program_id(0) == 0)
    def _():
      st_ref[...] = tile

    @pl.when(pl.program_id(0) != 0)
    def _():
      st_ref[...] += tile

  return pl.pallas_call(
      body,
      grid=(NB,),
      in_specs=[pl.BlockSpec((BLK, C), lambda b: (b, 0))],
      out_specs=pl.BlockSpec((8, 128), lambda b: (0, 0)),
      out_shape=jax.ShapeDtypeStruct((8, 128), jnp.float32),
      compiler_params=pltpu.CompilerParams(
          dimension_semantics=("arbitrary",)),
  )(out_pre)


def _tc_bn_relu(out_pre, stats, gamma8, beta8):
  def body(o_ref, st_ref, ga_ref, be_ref, out_ref):
    s = st_ref[0:1, 0:C]
    q = st_ref[1:2, 0:C]
    mean = s * (1.0 / N)
    var = q * (1.0 / N) - mean * mean
    inv = lax.rsqrt(var + 1e-5)
    scale = ga_ref[0:1, :] * inv
    shift = be_ref[0:1, :] - mean * scale
    out_ref[...] = jnp.maximum(o_ref[...] * scale + shift, 0.0)

  return pl.pallas_call(
      body,
      grid=(NB,),
      in_specs=[
          pl.BlockSpec((BLK, C), lambda b: (b, 0)),
          pl.BlockSpec((8, 128), lambda b: (0, 0)),
          pl.BlockSpec((8, C), lambda b: (0, 0)),
          pl.BlockSpec((8, C), lambda b: (0, 0)),
      ],
      out_specs=pl.BlockSpec((BLK, C), lambda b: (b, 0)),
      out_shape=jax.ShapeDtypeStruct((NP, C), jnp.float32),
      compiler_params=pltpu.CompilerParams(
          dimension_semantics=("parallel",)),
  )(out_pre, stats, gamma8, beta8)


def kernel(feats, W, gamma, beta, src, dst):
  feats_pad = jnp.concatenate(
      [feats, jnp.zeros((8, C), jnp.float32)], axis=0)
  src_flat = src.reshape(K * N)
  dst_flat = dst.reshape(K * N)
  wflat = W.reshape(K * C, C)
  gamma8 = jnp.broadcast_to(gamma[None, :], (8, C))
  beta8 = jnp.broadcast_to(beta[None, :], (8, C))

  g = _sc_invert_gather(feats_pad, src_flat, dst_flat)
  out_pre = _tc_gemm(g, wflat)
  stats = _tc_stats(out_pre)
  out = _tc_bn_relu(out_pre, stats, gamma8, beta8)
  return out[:N]
